# Initial kernel scaffold; baseline (speedup 1.0000x reference)
import jax
import jax.numpy as jnp
from jax.experimental import pallas as pl


def kernel(conv1_w, conv1_b, conv2_w, conv2_b, conv3_w, conv3_b, conv4_w, conv4_b, conv5_w, conv5_b, fc1_w, fc1_b, fc2_w, fc2_b, fc3_w, fc3_b, x_nchw):
    raise NotImplementedError("write your pallas kernel here")


# speedup vs baseline: 37.3418x; 37.3418x over previous
import jax
import jax.numpy as jnp
from jax.experimental import pallas as pl
from jax.experimental.pallas import tpu as pltpu


def _im2col(x, kh, kw, ho, wo):
    c = x.shape[-1]
    pieces = [x[di:di + ho, dj:dj + wo, :]
              for di in range(kh) for dj in range(kw)]
    p = jnp.concatenate(pieces, axis=-1)
    return p.reshape(ho * wo, kh * kw * c)


def _pad_hw(x, p):
    return jnp.pad(x, ((p, p), (p, p), (0, 0)))


def _pool3x3s2(y):
    h, w, c = y.shape
    ho = (h - 3) // 2 + 1
    wo = (w - 3) // 2 + 1
    z = jnp.concatenate([y, y[:1]], axis=0).reshape(ho + 1, 2, w, c)
    s0 = z[:, 0]
    s1 = z[:, 1]
    y = jnp.maximum(jnp.maximum(s0[:ho], s1[:ho]), s0[1:ho + 1])
    z = jnp.concatenate([y, y[:, :1]], axis=1).reshape(ho, wo + 1, 2, c)
    s0 = z[:, :, 0]
    s1 = z[:, :, 1]
    return jnp.maximum(jnp.maximum(s0[:, :wo], s1[:, :wo]), s0[:, 1:wo + 1])


def _conv_block(x, w_ref, b_ref, kh, kw, ho, wo):
    p = _im2col(x, kh, kw, ho, wo)
    acc = jnp.dot(p, w_ref[...], preferred_element_type=jnp.float32)
    y = jnp.maximum(acc + b_ref[...], 0.0).astype(jnp.bfloat16)
    return y.reshape(ho, wo, w_ref.shape[-1])


def _l1_kernel(x_ref, w_ref, b_ref, o_ref):
    y = _conv_block(x_ref[0], w_ref, b_ref, 3, 3, 55, 55)
    o_ref[0] = _pool3x3s2(y)


def _l2_kernel(x_ref, w_ref, b_ref, o_ref):
    x = _pad_hw(x_ref[0], 2)
    y = _conv_block(x, w_ref, b_ref, 5, 5, 27, 27)
    o_ref[0] = _pool3x3s2(y)


def _l345_kernel(x_ref, w3_ref, b3_ref, w4_ref, b4_ref, w5_ref, b5_ref,
                 o_ref):
    x = _pad_hw(x_ref[0], 1)
    y = _conv_block(x, w3_ref, b3_ref, 3, 3, 13, 13)
    y = _pad_hw(y, 1)
    y = _conv_block(y, w4_ref, b4_ref, 3, 3, 13, 13)
    y = _pad_hw(y, 1)
    y = _conv_block(y, w5_ref, b5_ref, 3, 3, 13, 13)
    o_ref[0] = _pool3x3s2(y)


def _fc_kernel(x_ref, w1_ref, b1_ref, w2_ref, b2_ref, w3_ref, b3_ref,
               o_ref):
    x = x_ref[...]
    h = jnp.concatenate(
        [jnp.dot(x, w1_ref[j], preferred_element_type=jnp.float32)
         for j in range(w1_ref.shape[0])], axis=1)
    h = jnp.maximum(h + b1_ref[...], 0.0).astype(jnp.bfloat16)
    h = jnp.concatenate(
        [jnp.dot(h, w2_ref[j], preferred_element_type=jnp.float32)
         for j in range(w2_ref.shape[0])], axis=1)
    h = jnp.maximum(h + b2_ref[...], 0.0).astype(jnp.bfloat16)
    o = jnp.concatenate(
        [jnp.dot(h, w3_ref[j], preferred_element_type=jnp.float32)
         for j in range(w3_ref.shape[0])], axis=1)
    o_ref[...] = o + b3_ref[...]


def _full_spec(shape):
    nd = len(shape)
    return pl.BlockSpec(shape, lambda n, _nd=nd: (0,) * _nd)


def _per_image(shape):
    nd = len(shape)
    return pl.BlockSpec((1,) + shape[1:],
                        lambda n, _nd=nd: (n,) + (0,) * (_nd - 1))


def _conv_layer(body, x, weights, out_shape):
    n = x.shape[0]
    specs = [_per_image(x.shape)]
    for wgt in weights:
        specs.append(_full_spec(wgt.shape))
    return pl.pallas_call(
        body,
        out_shape=jax.ShapeDtypeStruct((n,) + out_shape, jnp.bfloat16),
        grid=(n,),
        in_specs=specs,
        out_specs=_per_image((n,) + out_shape),
        compiler_params=pltpu.CompilerParams(
            dimension_semantics=("parallel",),
            vmem_limit_bytes=96 * 1024 * 1024),
    )(x, *weights)


def kernel(conv1_w, conv1_b, conv2_w, conv2_b, conv3_w, conv3_b,
           conv4_w, conv4_b, conv5_w, conv5_b, fc1_w, fc1_b,
           fc2_w, fc2_b, fc3_w, fc3_b, x_nchw):
    n = x_nchw.shape[0]
    x = jnp.transpose(x_nchw, (0, 2, 3, 1)).astype(jnp.bfloat16)
    xp = jnp.pad(x, ((0, 0), (2, 2), (2, 2), (0, 0)))
    xs = (xp.reshape(n, 57, 4, 57, 4, 3)
          .transpose(0, 1, 3, 2, 4, 5)
          .reshape(n, 57, 57, 48))
    xs = jnp.pad(xs, ((0, 0), (0, 0), (0, 0), (0, 16)))

    w1 = jnp.pad(conv1_w[0, :432].reshape(3, 3, 48, 64),
                 ((0, 0), (0, 0), (0, 16), (0, 0))).reshape(576, 64)
    w2 = conv2_w[0, :1600]
    w3 = conv3_w[0]
    w4 = conv4_w[0]
    w5 = conv5_w[0]

    y = _conv_layer(_l1_kernel, xs, (w1, conv1_b), (27, 27, 64))
    y = _conv_layer(_l2_kernel, y, (w2, conv2_b), (13, 13, 128))
    y = _conv_layer(_l345_kernel, y,
                    (w3, conv3_b, w4, conv4_b, w5, conv5_b), (6, 6, 128))

    xf = jnp.transpose(y, (0, 3, 1, 2)).reshape(n, 4608)
    out = pl.pallas_call(
        _fc_kernel,
        out_shape=jax.ShapeDtypeStruct((n, 1024), jnp.float32),
        grid=(1,),
        in_specs=[_full_spec(xf.shape), _full_spec(fc1_w.shape),
                  _full_spec(fc1_b.shape), _full_spec(fc2_w.shape),
                  _full_spec(fc2_b.shape), _full_spec(fc3_w.shape),
                  _full_spec(fc3_b.shape)],
        out_specs=_full_spec((n, 1024)),
        compiler_params=pltpu.CompilerParams(
            dimension_semantics=("arbitrary",),
            vmem_limit_bytes=96 * 1024 * 1024),
    )(xf, fc1_w, fc1_b, fc2_w, fc2_b, fc3_w, fc3_b)
    return out[:, :1000]


# speedup vs baseline: 60.6768x; 1.6249x over previous
import jax
import jax.numpy as jnp
from jax.experimental import pallas as pl
from jax.experimental.pallas import tpu as pltpu


def _im2col(x, kh, kw, ho, wo):
    c = x.shape[-1]
    pieces = [x[di:di + ho, dj:dj + wo, :]
              for di in range(kh) for dj in range(kw)]
    p = jnp.concatenate(pieces, axis=-1)
    return p.reshape(ho * wo, kh * kw * c)


def _pool3x3s2(y, ho, wo):
    h, w, c = y.shape
    p = ho + 1
    if h < 2 * p:
        y = jnp.concatenate([y, y[:2 * p - h]], axis=0)
    z = y.reshape(p, 2, w, c)
    s0 = z[:, 0]
    s1 = z[:, 1]
    y = jnp.maximum(jnp.maximum(s0[:ho], s1[:ho]), s0[1:p])
    q = wo + 1
    z = y[:, :2 * q].reshape(ho, q, 2, c)
    s0 = z[:, :, 0]
    s1 = z[:, :, 1]
    return jnp.maximum(jnp.maximum(s0[:, :wo], s1[:, :wo]), s0[:, 1:q])


def _conv_block(x, w_ref, b_ref, kh, kw, ho, wo):
    p = _im2col(x, kh, kw, ho, wo)
    acc = jnp.dot(p, w_ref[...], preferred_element_type=jnp.float32)
    y = jnp.maximum(acc + b_ref[...], 0.0).astype(jnp.bfloat16)
    return y.reshape(ho, wo, w_ref.shape[-1])


def _zero_cols(y, keep):
    wmask = jax.lax.broadcasted_iota(jnp.int32, (1, y.shape[1], 1), 1) < keep
    return jnp.where(wmask, y, jnp.bfloat16(0.0))


def _pad_lr(x, left, right):
    h, _, c = x.shape
    zl = jnp.zeros((h, left, c), x.dtype)
    zr = jnp.zeros((h, right, c), x.dtype)
    return jnp.concatenate([zl, x, zr], axis=1)


def _pad_tb(x, top, bottom):
    _, w, c = x.shape
    zt = jnp.zeros((top, w, c), x.dtype)
    zb = jnp.zeros((bottom, w, c), x.dtype)
    return jnp.concatenate([zt, x, zb], axis=0)


def _l1_kernel(x_ref, w_ref, b_ref, o_ref):
    y = _conv_block(x_ref[0], w_ref, b_ref, 3, 3, 55, 56)
    o_ref[0] = _pool3x3s2(y, 27, 27)


def _l2_kernel(x_ref, w_ref, b_ref, o_ref):
    x = _pad_tb(_pad_lr(x_ref[0], 2, 7), 2, 2)
    y = _conv_block(x, w_ref, b_ref, 5, 5, 27, 32)
    o_ref[0] = _pool3x3s2(y, 13, 13)


def _l345_kernel(x_ref, w3_ref, b3_ref, w4_ref, b4_ref, w5_ref, b5_ref,
                 o_ref):
    x = _pad_tb(_pad_lr(x_ref[0], 1, 4), 1, 1)
    y = _conv_block(x, w3_ref, b3_ref, 3, 3, 13, 16)
    x = _pad_tb(_pad_lr(_zero_cols(y, 13), 1, 1), 1, 1)
    y = _conv_block(x, w4_ref, b4_ref, 3, 3, 13, 16)
    x = _pad_tb(_pad_lr(_zero_cols(y, 13), 1, 1), 1, 1)
    y = _conv_block(x, w5_ref, b5_ref, 3, 3, 13, 16)
    o_ref[0] = _pool3x3s2(y, 6, 6)


def _fc_kernel(x_ref, w1_ref, b1_ref, w2_ref, b2_ref, w3_ref, b3_ref,
               o_ref):
    x = x_ref[...]
    h = jnp.concatenate(
        [jnp.dot(x, w1_ref[j], preferred_element_type=jnp.float32)
         for j in range(w1_ref.shape[0])], axis=1)
    h = jnp.maximum(h + b1_ref[...], 0.0).astype(jnp.bfloat16)
    h = jnp.concatenate(
        [jnp.dot(h, w2_ref[j], preferred_element_type=jnp.float32)
         for j in range(w2_ref.shape[0])], axis=1)
    h = jnp.maximum(h + b2_ref[...], 0.0).astype(jnp.bfloat16)
    o = jnp.concatenate(
        [jnp.dot(h, w3_ref[j], preferred_element_type=jnp.float32)
         for j in range(w3_ref.shape[0])], axis=1)
    o_ref[...] = o + b3_ref[...]


def _full_spec(shape):
    nd = len(shape)
    return pl.BlockSpec(shape, lambda *_, _nd=nd: (0,) * _nd)


def _per_image(shape):
    nd = len(shape)
    return pl.BlockSpec((1,) + shape[1:],
                        lambda n, _nd=nd: (n,) + (0,) * (_nd - 1))


def _conv_layer(body, x, weights, out_shape):
    n = x.shape[0]
    specs = [_per_image(x.shape)]
    for wgt in weights:
        specs.append(_full_spec(wgt.shape))
    return pl.pallas_call(
        body,
        out_shape=jax.ShapeDtypeStruct((n,) + out_shape, jnp.bfloat16),
        grid=(n,),
        in_specs=specs,
        out_specs=_per_image((n,) + out_shape),
        compiler_params=pltpu.CompilerParams(
            dimension_semantics=("parallel",),
            vmem_limit_bytes=96 * 1024 * 1024),
    )(x, *weights)


def kernel(conv1_w, conv1_b, conv2_w, conv2_b, conv3_w, conv3_b,
           conv4_w, conv4_b, conv5_w, conv5_b, fc1_w, fc1_b,
           fc2_w, fc2_b, fc3_w, fc3_b, x_nchw):
    n = x_nchw.shape[0]
    x = jnp.transpose(x_nchw, (0, 2, 3, 1)).astype(jnp.bfloat16)
    xp = jnp.pad(x, ((0, 0), (2, 2), (2, 2), (0, 0)))
    xs = (xp.reshape(n, 57, 4, 57, 4, 3)
          .transpose(0, 1, 3, 2, 4, 5)
          .reshape(n, 57, 57, 48))
    xs = jnp.pad(xs, ((0, 0), (0, 0), (0, 1), (0, 16)))

    w1 = jnp.pad(conv1_w[0, :432].reshape(3, 3, 48, 64),
                 ((0, 0), (0, 0), (0, 16), (0, 0))).reshape(576, 64)
    w2 = conv2_w[0, :1600]
    w3 = conv3_w[0]
    w4 = conv4_w[0]
    w5 = conv5_w[0]

    y = _conv_layer(_l1_kernel, xs, (w1, conv1_b), (27, 27, 64))
    y = _conv_layer(_l2_kernel, y, (w2, conv2_b), (13, 13, 128))
    y = _conv_layer(_l345_kernel, y,
                    (w3, conv3_b, w4, conv4_b, w5, conv5_b), (6, 6, 128))

    xf = jnp.transpose(y, (0, 3, 1, 2)).reshape(n, 4608)
    out = pl.pallas_call(
        _fc_kernel,
        out_shape=jax.ShapeDtypeStruct((n, 1024), jnp.float32),
        grid=(1,),
        in_specs=[_full_spec(xf.shape), _full_spec(fc1_w.shape),
                  _full_spec(fc1_b.shape), _full_spec(fc2_w.shape),
                  _full_spec(fc2_b.shape), _full_spec(fc3_w.shape),
                  _full_spec(fc3_b.shape)],
        out_specs=_full_spec((n, 1024)),
        compiler_params=pltpu.CompilerParams(
            dimension_semantics=("arbitrary",),
            vmem_limit_bytes=96 * 1024 * 1024),
    )(xf, fc1_w, fc1_b, fc2_w, fc2_b, fc3_w, fc3_b)
    return out[:, :1000]


# speedup vs baseline: 64.4958x; 1.0629x over previous
import jax
import jax.numpy as jnp
from jax.experimental import pallas as pl
from jax.experimental.pallas import tpu as pltpu


def _im2col(x, kh, kw, ho, wo):
    c = x.shape[-1]
    pieces = [x[di:di + ho, dj:dj + wo, :]
              for di in range(kh) for dj in range(kw)]
    p = jnp.concatenate(pieces, axis=-1)
    return p.reshape(ho * wo, kh * kw * c)


def _pool3x3s2(y, ho, wo):
    h, w, c = y.shape
    p = ho + 1
    if h < 2 * p:
        y = jnp.concatenate([y, y[:2 * p - h]], axis=0)
    z = y.reshape(p, 2, w, c)
    s0 = z[:, 0]
    s1 = z[:, 1]
    y = jnp.maximum(jnp.maximum(s0[:ho], s1[:ho]), s0[1:p])
    q = wo + 1
    z = y[:, :2 * q].reshape(ho, q, 2, c)
    s0 = z[:, :, 0]
    s1 = z[:, :, 1]
    return jnp.maximum(jnp.maximum(s0[:, :wo], s1[:, :wo]), s0[:, 1:q])


def _conv_block(x, w_ref, b_ref, kh, kw, ho, wo):
    p = _im2col(x, kh, kw, ho, wo)
    acc = jnp.dot(p, w_ref[...], preferred_element_type=jnp.float32)
    y = jnp.maximum(acc + b_ref[...], 0.0).astype(jnp.bfloat16)
    return y.reshape(ho, wo, w_ref.shape[-1])


def _zero_cols(y, keep):
    wmask = jax.lax.broadcasted_iota(jnp.int32, (1, y.shape[1], 1), 1) < keep
    return jnp.where(wmask, y, jnp.bfloat16(0.0))


def _pad_lr(x, left, right):
    h, _, c = x.shape
    zl = jnp.zeros((h, left, c), x.dtype)
    zr = jnp.zeros((h, right, c), x.dtype)
    return jnp.concatenate([zl, x, zr], axis=1)


def _pad_tb(x, top, bottom):
    _, w, c = x.shape
    zt = jnp.zeros((top, w, c), x.dtype)
    zb = jnp.zeros((bottom, w, c), x.dtype)
    return jnp.concatenate([zt, x, zb], axis=0)


def _l1_kernel(x_ref, w_ref, b_ref, o_ref):
    y = _conv_block(x_ref[0], w_ref, b_ref, 3, 3, 55, 56)
    o_ref[0] = _pool3x3s2(y, 27, 27)


def _l2_kernel(x_ref, w_ref, b_ref, o_ref):
    x = _pad_tb(_pad_lr(x_ref[0], 2, 7), 2, 2)
    y = _conv_block(x, w_ref, b_ref, 5, 5, 27, 32)
    o_ref[0] = _pool3x3s2(y, 13, 13)


def _l345_kernel(x_ref, w3_ref, b3_ref, w4_ref, b4_ref, w5_ref, b5_ref,
                 o_ref):
    x = _pad_tb(_pad_lr(x_ref[0], 1, 4), 1, 1)
    y = _conv_block(x, w3_ref, b3_ref, 3, 3, 13, 16)
    x = _pad_tb(_pad_lr(_zero_cols(y, 13), 1, 1), 1, 1)
    y = _conv_block(x, w4_ref, b4_ref, 3, 3, 13, 16)
    x = _pad_tb(_pad_lr(_zero_cols(y, 13), 1, 1), 1, 1)
    y = _conv_block(x, w5_ref, b5_ref, 3, 3, 13, 16)
    o_ref[0] = _pool3x3s2(y, 6, 6)


def _fc_kernel(x_ref, w1_ref, b1_ref, w2_ref, b2_ref, w3_ref, b3_ref,
               o_ref):
    x = x_ref[...]
    h = jnp.concatenate(
        [jnp.dot(x, w1_ref[j], preferred_element_type=jnp.float32)
         for j in range(w1_ref.shape[0])], axis=1)
    h = jnp.maximum(h + b1_ref[...], 0.0).astype(jnp.bfloat16)
    h = jnp.concatenate(
        [jnp.dot(h, w2_ref[j], preferred_element_type=jnp.float32)
         for j in range(w2_ref.shape[0])], axis=1)
    h = jnp.maximum(h + b2_ref[...], 0.0).astype(jnp.bfloat16)
    o = jnp.concatenate(
        [jnp.dot(h, w3_ref[j], preferred_element_type=jnp.float32)
         for j in range(w3_ref.shape[0])], axis=1)
    o_ref[...] = o + b3_ref[...]


def _full_spec(shape):
    nd = len(shape)
    return pl.BlockSpec(shape, lambda *_, _nd=nd: (0,) * _nd)


def _per_image(shape):
    nd = len(shape)
    return pl.BlockSpec((1,) + shape[1:],
                        lambda n, _nd=nd: (n,) + (0,) * (_nd - 1))


def _conv_layer(body, x, weights, out_shape):
    n = x.shape[0]
    specs = [_per_image(x.shape)]
    for wgt in weights:
        specs.append(_full_spec(wgt.shape))
    return pl.pallas_call(
        body,
        out_shape=jax.ShapeDtypeStruct((n,) + out_shape, jnp.bfloat16),
        grid=(n,),
        in_specs=specs,
        out_specs=_per_image((n,) + out_shape),
        compiler_params=pltpu.CompilerParams(
            dimension_semantics=("parallel",),
            allow_input_fusion=[True] + [False] * len(weights),
            vmem_limit_bytes=96 * 1024 * 1024),
    )(x, *weights)


def kernel(conv1_w, conv1_b, conv2_w, conv2_b, conv3_w, conv3_b,
           conv4_w, conv4_b, conv5_w, conv5_b, fc1_w, fc1_b,
           fc2_w, fc2_b, fc3_w, fc3_b, x_nchw):
    n = x_nchw.shape[0]
    xp = jnp.pad(x_nchw.astype(jnp.bfloat16),
                 ((0, 0), (0, 0), (2, 2), (2, 2)))
    xs = (xp.reshape(n, 3, 57, 4, 57, 4)
          .transpose(0, 2, 4, 3, 5, 1)
          .reshape(n, 57, 57, 48))
    xs = jnp.pad(xs, ((0, 0), (0, 0), (0, 1), (0, 16)))

    w1 = jnp.pad(conv1_w[0, :432].reshape(3, 3, 48, 64),
                 ((0, 0), (0, 0), (0, 16), (0, 0))).reshape(576, 64)
    w2 = conv2_w[0, :1600]
    w3 = conv3_w[0]
    w4 = conv4_w[0]
    w5 = conv5_w[0]

    y = _conv_layer(_l1_kernel, xs, (w1, conv1_b), (27, 27, 64))
    y = _conv_layer(_l2_kernel, y, (w2, conv2_b), (13, 13, 128))
    y = _conv_layer(_l345_kernel, y,
                    (w3, conv3_b, w4, conv4_b, w5, conv5_b), (6, 6, 128))

    xf = jnp.transpose(y, (0, 3, 1, 2)).reshape(n, 4608)
    out = pl.pallas_call(
        _fc_kernel,
        out_shape=jax.ShapeDtypeStruct((n, 1024), jnp.float32),
        grid=(1,),
        in_specs=[_full_spec(xf.shape), _full_spec(fc1_w.shape),
                  _full_spec(fc1_b.shape), _full_spec(fc2_w.shape),
                  _full_spec(fc2_b.shape), _full_spec(fc3_w.shape),
                  _full_spec(fc3_b.shape)],
        out_specs=_full_spec((n, 1024)),
        compiler_params=pltpu.CompilerParams(
            dimension_semantics=("arbitrary",),
            vmem_limit_bytes=96 * 1024 * 1024),
    )(xf, fc1_w, fc1_b, fc2_w, fc2_b, fc3_w, fc3_b)
    return out[:, :1000]


# speedup vs baseline: 70.1915x; 1.0883x over previous
import jax
import jax.numpy as jnp
from jax.experimental import pallas as pl
from jax.experimental.pallas import tpu as pltpu


def _im2col(x, kh, kw, ho, wo):
    c = x.shape[-1]
    pieces = [x[di:di + ho, dj:dj + wo, :]
              for di in range(kh) for dj in range(kw)]
    p = jnp.concatenate(pieces, axis=-1)
    return p.reshape(ho * wo, kh * kw * c)


def _pool3x3s2(y, ho, wo):
    h, w, c = y.shape
    p = ho + 1
    if h < 2 * p:
        y = jnp.concatenate([y, y[:2 * p - h]], axis=0)
    z = y.reshape(p, 2, w, c)
    s0 = z[:, 0]
    s1 = z[:, 1]
    y = jnp.maximum(jnp.maximum(s0[:ho], s1[:ho]), s0[1:p])
    q = wo + 1
    z = y[:, :2 * q].reshape(ho, q, 2, c)
    s0 = z[:, :, 0]
    s1 = z[:, :, 1]
    return jnp.maximum(jnp.maximum(s0[:, :wo], s1[:, :wo]), s0[:, 1:q])


def _conv_block(x, w_ref, b_ref, kh, kw, ho, wo):
    p = _im2col(x, kh, kw, ho, wo)
    acc = jnp.dot(p, w_ref[...], preferred_element_type=jnp.float32)
    y = jnp.maximum(acc + b_ref[...], 0.0).astype(jnp.bfloat16)
    return y.reshape(ho, wo, w_ref.shape[-1])


def _zero_cols(y, keep):
    wmask = jax.lax.broadcasted_iota(jnp.int32, (1, y.shape[1], 1), 1) < keep
    return jnp.where(wmask, y, jnp.bfloat16(0.0))


def _pad_lr(x, left, right):
    h, _, c = x.shape
    zl = jnp.zeros((h, left, c), x.dtype)
    zr = jnp.zeros((h, right, c), x.dtype)
    return jnp.concatenate([zl, x, zr], axis=1)


def _pad_tb(x, top, bottom):
    _, w, c = x.shape
    zt = jnp.zeros((top, w, c), x.dtype)
    zb = jnp.zeros((bottom, w, c), x.dtype)
    return jnp.concatenate([zt, x, zb], axis=0)


def _l1_kernel(x_ref, w_ref, b_ref, o_ref):
    y = _conv_block(x_ref[0], w_ref, b_ref, 3, 3, 55, 56)
    o_ref[0] = _pool3x3s2(y, 27, 27)


def _l2_kernel(x_ref, w_ref, b_ref, o_ref):
    x = _pad_tb(_pad_lr(x_ref[0], 2, 7), 2, 2)
    y = _conv_block(x, w_ref, b_ref, 5, 5, 27, 32)
    o_ref[0] = _pool3x3s2(y, 13, 13)


def _pad_hw_b(x, p):
    h, w, b, c = x.shape
    zw = jnp.zeros((h, p, b, c), x.dtype)
    x = jnp.concatenate([zw, x, zw], axis=1)
    zh = jnp.zeros((p, w + 2 * p, b, c), x.dtype)
    return jnp.concatenate([zh, x, zh], axis=0)


def _conv_block_b(x, w_ref, b_ref, kh, kw, ho, wo):
    _, _, b, c = x.shape
    pieces = [x[di:di + ho, dj:dj + wo]
              for di in range(kh) for dj in range(kw)]
    p = jnp.concatenate(pieces, axis=-1).reshape(ho * wo * b, kh * kw * c)
    acc = jnp.dot(p, w_ref[...], preferred_element_type=jnp.float32)
    y = jnp.maximum(acc + b_ref[...], 0.0).astype(jnp.bfloat16)
    return y.reshape(ho, wo, b, w_ref.shape[-1])


def _pool3x3s2_b(y, ho, wo):
    p, q = ho + 1, wo + 1
    if y.shape[0] < 2 * p:
        y = jnp.concatenate([y, y[:2 * p - y.shape[0]]], axis=0)
    z = y.reshape(p, 2, *y.shape[1:])
    s0, s1 = z[:, 0], z[:, 1]
    y = jnp.maximum(jnp.maximum(s0[:ho], s1[:ho]), s0[1:p])
    if y.shape[1] < 2 * q:
        y = jnp.concatenate([y, y[:, :2 * q - y.shape[1]]], axis=1)
    z = y.reshape(ho, q, 2, *y.shape[2:])
    s0, s1 = z[:, :, 0], z[:, :, 1]
    return jnp.maximum(jnp.maximum(s0[:, :wo], s1[:, :wo]), s0[:, 1:q])


def _l345_kernel(x_ref, w3_ref, b3_ref, w4_ref, b4_ref, w5_ref, b5_ref,
                 o_ref):
    x = _pad_hw_b(x_ref[0], 1)
    y = _conv_block_b(x, w3_ref, b3_ref, 3, 3, 13, 13)
    y = _conv_block_b(_pad_hw_b(y, 1), w4_ref, b4_ref, 3, 3, 13, 13)
    y = _conv_block_b(_pad_hw_b(y, 1), w5_ref, b5_ref, 3, 3, 13, 13)
    o_ref[0] = _pool3x3s2_b(y, 6, 6)


def _fc_kernel(x_ref, w1_ref, b1_ref, w2_ref, b2_ref, w3_ref, b3_ref,
               o_ref):
    x = x_ref[...]
    h = jnp.concatenate(
        [jnp.dot(x, w1_ref[j], preferred_element_type=jnp.float32)
         for j in range(w1_ref.shape[0])], axis=1)
    h = jnp.maximum(h + b1_ref[...], 0.0).astype(jnp.bfloat16)
    h = jnp.concatenate(
        [jnp.dot(h, w2_ref[j], preferred_element_type=jnp.float32)
         for j in range(w2_ref.shape[0])], axis=1)
    h = jnp.maximum(h + b2_ref[...], 0.0).astype(jnp.bfloat16)
    o = jnp.concatenate(
        [jnp.dot(h, w3_ref[j], preferred_element_type=jnp.float32)
         for j in range(w3_ref.shape[0])], axis=1)
    o_ref[...] = o + b3_ref[...]


def _full_spec(shape):
    nd = len(shape)
    return pl.BlockSpec(shape, lambda *_, _nd=nd: (0,) * _nd)


def _per_image(shape):
    nd = len(shape)
    return pl.BlockSpec((1,) + shape[1:],
                        lambda n, _nd=nd: (n,) + (0,) * (_nd - 1))


def _conv_layer(body, x, weights, out_shape):
    n = x.shape[0]
    specs = [_per_image(x.shape)]
    for wgt in weights:
        specs.append(_full_spec(wgt.shape))
    return pl.pallas_call(
        body,
        out_shape=jax.ShapeDtypeStruct((n,) + out_shape, jnp.bfloat16),
        grid=(n,),
        in_specs=specs,
        out_specs=_per_image((n,) + out_shape),
        compiler_params=pltpu.CompilerParams(
            dimension_semantics=("parallel",),
            allow_input_fusion=[True] + [False] * len(weights),
            vmem_limit_bytes=96 * 1024 * 1024),
    )(x, *weights)


def kernel(conv1_w, conv1_b, conv2_w, conv2_b, conv3_w, conv3_b,
           conv4_w, conv4_b, conv5_w, conv5_b, fc1_w, fc1_b,
           fc2_w, fc2_b, fc3_w, fc3_b, x_nchw):
    n = x_nchw.shape[0]
    xp = jnp.pad(x_nchw.astype(jnp.bfloat16),
                 ((0, 0), (0, 0), (2, 2), (2, 2)))
    xs = (xp.reshape(n, 3, 57, 4, 57, 4)
          .transpose(0, 2, 4, 3, 5, 1)
          .reshape(n, 57, 57, 48))
    xs = jnp.pad(xs, ((0, 0), (0, 0), (0, 1), (0, 16)))

    w1 = jnp.pad(conv1_w[0, :432].reshape(3, 3, 48, 64),
                 ((0, 0), (0, 0), (0, 16), (0, 0))).reshape(576, 64)
    w2 = conv2_w[0, :1600]
    w3 = conv3_w[0]
    w4 = conv4_w[0]
    w5 = conv5_w[0]

    y = _conv_layer(_l1_kernel, xs, (w1, conv1_b), (27, 27, 64))
    y = _conv_layer(_l2_kernel, y, (w2, conv2_b), (13, 13, 128))
    nb = n // 16
    yb = y.reshape(nb, 16, 13, 13, 128).transpose(0, 2, 3, 1, 4)
    y = _conv_layer(_l345_kernel, yb,
                    (w3, conv3_b, w4, conv4_b, w5, conv5_b),
                    (6, 6, 16, 128))

    xf = jnp.transpose(y, (0, 3, 4, 1, 2)).reshape(n, 4608)
    out = pl.pallas_call(
        _fc_kernel,
        out_shape=jax.ShapeDtypeStruct((n, 1024), jnp.float32),
        grid=(1,),
        in_specs=[_full_spec(xf.shape), _full_spec(fc1_w.shape),
                  _full_spec(fc1_b.shape), _full_spec(fc2_w.shape),
                  _full_spec(fc2_b.shape), _full_spec(fc3_w.shape),
                  _full_spec(fc3_b.shape)],
        out_specs=_full_spec((n, 1024)),
        compiler_params=pltpu.CompilerParams(
            dimension_semantics=("arbitrary",),
            vmem_limit_bytes=96 * 1024 * 1024),
    )(xf, fc1_w, fc1_b, fc2_w, fc2_b, fc3_w, fc3_b)
    return out[:, :1000]
